# Initial kernel scaffold; baseline (speedup 1.0000x reference)
import functools

import jax
import jax.numpy as jnp
from jax import lax
from jax.experimental import pallas as pl
from jax.experimental.pallas import tpu as pltpu
from jax.experimental.pallas import tpu_sc as plsc

N_NODES = 10000
N_EDGES = 320000
DIM = 128
CLIP = 511

NC, NS, L = 2, 16, 16
HIST = 10240
SPAN = HIST // NS
ECOLS = 128
EROWS = 2512
ROWS_PER_TILE = EROWS // NS
PAD_IDX = N_NODES

_mesh = plsc.VectorSubcoreMesh(
    core_axis_name="c", subcore_axis_name="s", num_cores=NC, num_subcores=NS
)


@functools.partial(
    pl.kernel,
    out_type=jax.ShapeDtypeStruct((2, HIST), jnp.int32),
    mesh=_mesh,
    scratch_types=[
        pltpu.VMEM((ROWS_PER_TILE, ECOLS), jnp.int32),
        pltpu.VMEM((ECOLS,), jnp.int32),
        pltpu.VMEM((SPAN,), jnp.int32),
        pltpu.VMEM_SHARED((HIST,), jnp.int32),
    ],
)
def _degrees(ei_hbm, deg_hbm, idx_all, ones_v, deg_v, hist_sp):
    c = lax.axis_index("c")
    s = lax.axis_index("s")

    one16 = jnp.full((L,), 1, jnp.int32)
    zero16 = jnp.zeros((L,), jnp.int32)
    for k in range(ECOLS // L):
        ones_v[pl.ds(k * L, L)] = one16
    for k in range(SPAN // L):
        deg_v[pl.ds(k * L, L)] = zero16

    off = pl.multiple_of(s * SPAN, 8)
    pltpu.sync_copy(deg_v, hist_sp.at[pl.ds(off, SPAN)])
    plsc.subcore_barrier()

    pltpu.sync_copy(ei_hbm.at[c, pl.ds(s * ROWS_PER_TILE, ROWS_PER_TILE), :],
                    idx_all)

    @pl.loop(0, ROWS_PER_TILE)
    def _(r):
        pltpu.sync_copy(ones_v, hist_sp.at[idx_all.at[r]], add=True)

    plsc.subcore_barrier()

    pltpu.sync_copy(hist_sp.at[pl.ds(off, SPAN)], deg_v)
    for k in range(SPAN // L):
        sl = pl.ds(k * L, L)
        deg_v[sl] = jnp.minimum(deg_v[sl], CLIP)
    pltpu.sync_copy(deg_v, deg_hbm.at[c, pl.ds(off, SPAN)])


CHUNK = 64
PER_TILE = 320


@functools.partial(
    pl.kernel,
    out_type=jax.ShapeDtypeStruct((N_NODES, DIM), jnp.float32),
    mesh=_mesh,
    scratch_types=[
        pltpu.VMEM((CHUNK,), jnp.int32),
        pltpu.VMEM((CHUNK,), jnp.int32),
        pltpu.VMEM((CHUNK, DIM), jnp.float32),
        pltpu.VMEM((CHUNK, DIM), jnp.float32),
        pltpu.VMEM((CHUNK, DIM), jnp.float32),
        pltpu.SemaphoreType.DMA,
    ],
)
def _encode(x_hbm, deg_hbm, zin_hbm, zout_hbm, out_hbm,
            din_v, dout_v, xb, zib, zob, sem):
    c = lax.axis_index("c")
    s = lax.axis_index("s")
    w = s * NC + c

    zero16 = jnp.zeros((L,), jnp.int32)
    for k in range(CHUNK // L):
        din_v[pl.ds(k * L, L)] = zero16
        dout_v[pl.ds(k * L, L)] = zero16

    def do_chunk(n0, cn):
        n0 = pl.multiple_of(n0, 8)
        pltpu.sync_copy(deg_hbm.at[1, pl.ds(n0, cn)], din_v.at[pl.ds(0, cn)])
        pltpu.sync_copy(deg_hbm.at[0, pl.ds(n0, cn)], dout_v.at[pl.ds(0, cn)])
        pltpu.sync_copy(x_hbm.at[pl.ds(n0, cn), :], xb.at[pl.ds(0, cn), :])
        cp1 = pltpu.async_copy(zin_hbm.at[din_v], zib, sem)
        cp2 = pltpu.async_copy(zout_hbm.at[dout_v], zob, sem)
        cp1.wait()
        cp2.wait()

        @pl.loop(0, cn)
        def _(i):
            for k in range(DIM // L):
                sl = pl.ds(k * L, L)
                xb[i, sl] = xb[i, sl] + zib[i, sl] + zob[i, sl]

        pltpu.sync_copy(xb.at[pl.ds(0, cn), :], out_hbm.at[pl.ds(n0, cn), :])

    nfull = jnp.where(w < 31, PER_TILE // CHUNK, 1)

    @pl.loop(0, nfull)
    def _(j):
        do_chunk(w * PER_TILE + j * CHUNK, CHUNK)

    @pl.when(w == 31)
    def _():
        do_chunk(jnp.int32(N_NODES - 16), 16)


def kernel(x, edge_index, z_in, z_out):
    pad = EROWS * ECOLS - N_EDGES
    ei = jnp.concatenate(
        [edge_index, jnp.full((2, pad), PAD_IDX, jnp.int32)], axis=1
    ).reshape(2, EROWS, ECOLS)
    deg = _degrees(ei)
    return _encode(x, deg, z_in, z_out)


# speedup vs baseline: 1.3342x; 1.3342x over previous
import functools

import jax
import jax.numpy as jnp
from jax import lax
from jax.experimental import pallas as pl
from jax.experimental.pallas import tpu as pltpu
from jax.experimental.pallas import tpu_sc as plsc

N_NODES = 10000
N_EDGES = 320000
DIM = 128
CLIP = 511

NC, NS, L = 2, 16, 16
HIST = 10240
SPAN = HIST // NS
ECOLS = 128
EROWS = 2512
ROWS_PER_TILE = EROWS // NS
PAD_IDX = N_NODES

_mesh = plsc.VectorSubcoreMesh(
    core_axis_name="c", subcore_axis_name="s", num_cores=NC, num_subcores=NS
)


_params = pltpu.CompilerParams(use_tc_tiling_on_sc=False)


@functools.partial(
    pl.kernel,
    out_type=jax.ShapeDtypeStruct((2, HIST), jnp.int32),
    mesh=_mesh,
    compiler_params=_params,
    scratch_types=[
        pltpu.VMEM((ROWS_PER_TILE, ECOLS), jnp.int32),
        pltpu.VMEM((ECOLS,), jnp.int32),
        pltpu.VMEM((SPAN,), jnp.int32),
        pltpu.VMEM_SHARED((HIST,), jnp.int32),
    ],
)
def _degrees(ei_hbm, deg_hbm, idx_all, ones_v, deg_v, hist_sp):
    c = lax.axis_index("c")
    s = lax.axis_index("s")

    one16 = jnp.full((L,), 1, jnp.int32)
    zero16 = jnp.zeros((L,), jnp.int32)
    for k in range(ECOLS // L):
        ones_v[pl.ds(k * L, L)] = one16
    for k in range(SPAN // L):
        deg_v[pl.ds(k * L, L)] = zero16

    off = pl.multiple_of(s * SPAN, 8)
    pltpu.sync_copy(deg_v, hist_sp.at[pl.ds(off, SPAN)])
    plsc.subcore_barrier()

    pltpu.sync_copy(ei_hbm.at[c, pl.ds(s * ROWS_PER_TILE, ROWS_PER_TILE), :],
                    idx_all)

    @pl.loop(0, ROWS_PER_TILE)
    def _(r):
        pltpu.sync_copy(ones_v, hist_sp.at[idx_all.at[r]], add=True)

    plsc.subcore_barrier()

    pltpu.sync_copy(hist_sp.at[pl.ds(off, SPAN)], deg_v)
    for k in range(SPAN // L):
        sl = pl.ds(k * L, L)
        deg_v[sl] = jnp.minimum(deg_v[sl], CLIP)
    pltpu.sync_copy(deg_v, deg_hbm.at[c, pl.ds(off, SPAN)])


CHUNK = 64
PER_TILE = 320


@functools.partial(
    pl.kernel,
    out_type=jax.ShapeDtypeStruct((N_NODES, DIM), jnp.float32),
    mesh=_mesh,
    compiler_params=_params,
    scratch_types=[
        pltpu.VMEM((CHUNK,), jnp.int32),
        pltpu.VMEM((CHUNK,), jnp.int32),
        pltpu.VMEM((CHUNK, DIM), jnp.float32),
        pltpu.VMEM((CHUNK, DIM), jnp.float32),
        pltpu.VMEM((CHUNK, DIM), jnp.float32),
        pltpu.SemaphoreType.DMA,
    ],
)
def _encode(x_hbm, deg_hbm, zin_hbm, zout_hbm, out_hbm,
            din_v, dout_v, xb, zib, zob, sem):
    c = lax.axis_index("c")
    s = lax.axis_index("s")
    w = s * NC + c

    zero16 = jnp.zeros((L,), jnp.int32)
    for k in range(CHUNK // L):
        din_v[pl.ds(k * L, L)] = zero16
        dout_v[pl.ds(k * L, L)] = zero16

    def do_chunk(n0, cn):
        n0 = pl.multiple_of(n0, 8)
        pltpu.sync_copy(deg_hbm.at[1, pl.ds(n0, cn)], din_v.at[pl.ds(0, cn)])
        pltpu.sync_copy(deg_hbm.at[0, pl.ds(n0, cn)], dout_v.at[pl.ds(0, cn)])
        pltpu.sync_copy(x_hbm.at[pl.ds(n0, cn), :], xb.at[pl.ds(0, cn), :])
        cp1 = pltpu.async_copy(zin_hbm.at[din_v], zib, sem)
        cp2 = pltpu.async_copy(zout_hbm.at[dout_v], zob, sem)
        cp1.wait()
        cp2.wait()

        @pl.loop(0, cn)
        def _(i):
            for k in range(DIM // L):
                sl = pl.ds(k * L, L)
                xb[i, sl] = xb[i, sl] + zib[i, sl] + zob[i, sl]

        pltpu.sync_copy(xb.at[pl.ds(0, cn), :], out_hbm.at[pl.ds(n0, cn), :])

    nfull = jnp.where(w < 31, PER_TILE // CHUNK, 1)

    @pl.loop(0, nfull)
    def _(j):
        do_chunk(w * PER_TILE + j * CHUNK, CHUNK)

    @pl.when(w == 31)
    def _():
        do_chunk(jnp.int32(N_NODES - 16), 16)


def kernel(x, edge_index, z_in, z_out):
    pad = EROWS * ECOLS - N_EDGES
    ei = jnp.concatenate(
        [edge_index, jnp.full((2, pad), PAD_IDX, jnp.int32)], axis=1
    ).reshape(2, EROWS, ECOLS)
    deg = _degrees(ei)
    return _encode(x, deg, z_in, z_out)
